# baseline (speedup 1.0000x reference)
import functools

import jax
import jax.numpy as jnp
from jax import lax
from jax.experimental import pallas as pl
from jax.experimental.pallas import tpu as pltpu
from jax.experimental.pallas import tpu_sc as plsc

N = 10000
E = 320000
D = 128
NPAD = 10240
NW = 32
EW = E // NW
NBLK = 81
EWP = NBLK * 128
NW1 = 16
EH = E // 2
EW1 = EH // NW1
MB = 128
SB = 8
NCH = 10
EWP1 = NCH * SB * MB
NSUB = NCH * SB
ROWS_PER_TILE = NPAD // 16

_MESH = plsc.VectorSubcoreMesh(core_axis_name="c", subcore_axis_name="s")
_MESH1 = plsc.VectorSubcoreMesh(core_axis_name="c", subcore_axis_name="s",
                                num_cores=1)


@functools.partial(
    pl.kernel,
    out_type=(jax.ShapeDtypeStruct((NPAD,), jnp.float32),
              jax.ShapeDtypeStruct((NPAD,), jnp.float32)),
    mesh=_MESH,
    scratch_types=[
        pltpu.VMEM((NBLK, 128), jnp.float32),
        pltpu.VMEM((NBLK, 128), jnp.int32),
        pltpu.VMEM((ROWS_PER_TILE,), jnp.float32),
        pltpu.VMEM_SHARED((NPAD,), jnp.float32),
    ],
)
def _deg_kernel(w_hbm, col_hbm, deg0_hbm, deg1_hbm, w_v, col_v, zb, acc):
    c = lax.axis_index("c")
    s = lax.axis_index("s")
    wid = s * 2 + c
    tb = s * ROWS_PER_TILE

    for i in range(ROWS_PER_TILE // 16):
        zb[pl.ds(i * 16, 16)] = jnp.zeros((16,), jnp.float32)
    pltpu.sync_copy(zb, acc.at[pl.ds(tb, ROWS_PER_TILE)])
    pltpu.sync_copy(w_hbm.at[wid], w_v)
    pltpu.sync_copy(col_hbm.at[wid], col_v)
    plsc.subcore_barrier()

    def body(j, carry):
        pltpu.sync_copy(w_v.at[j], acc.at[col_v.at[j]], add=True)
        return carry

    lax.fori_loop(0, NBLK, body, 0)
    plsc.subcore_barrier()

    @pl.when(c == 0)
    def _():
        pltpu.sync_copy(acc.at[pl.ds(tb, ROWS_PER_TILE)],
                        deg0_hbm.at[pl.ds(tb, ROWS_PER_TILE)])

    @pl.when(c == 1)
    def _():
        pltpu.sync_copy(acc.at[pl.ds(tb, ROWS_PER_TILE)],
                        deg1_hbm.at[pl.ds(tb, ROWS_PER_TILE)])


@functools.partial(
    pl.kernel,
    out_type=jax.ShapeDtypeStruct((NPAD, D), jnp.float32),
    mesh=_MESH1,
    scratch_types=[
        pltpu.VMEM((2, SB, MB), jnp.int32),
        pltpu.VMEM((2, SB, MB), jnp.int32),
        pltpu.VMEM((2 * SB * MB,), jnp.float32),
        pltpu.VMEM((MB, D), jnp.float32),
        pltpu.VMEM((MB, D), jnp.float32),
        pltpu.VMEM_SHARED((NPAD, D), jnp.float32),
        pltpu.SemaphoreType.DMA,
        pltpu.SemaphoreType.DMA,
        pltpu.SemaphoreType.DMA,
        pltpu.SemaphoreType.DMA,
        pltpu.SemaphoreType.DMA,
    ],
)
def _mp_kernel(g_hbm, row_hbm, col_hbm, w_hbm, p_hbm,
               row_r, col_r, w_r, m0, m1, acc,
               gs0, gs1, ss0, ss1, isem):
    s = lax.axis_index("s")
    wid = s
    tb = s * ROWS_PER_TILE
    bufs = (m0, m1)
    gsems = (gs0, gs1)
    ssems = (ss0, ss1)
    wchunk = SB * MB

    def zbody(r, carry):
        for k in range(8):
            m0[r, pl.ds(k * 16, 16)] = jnp.zeros((16,), jnp.float32)
        return carry

    lax.fori_loop(0, MB, zbody, 0)
    for r0 in range(ROWS_PER_TILE // MB):
        pltpu.sync_copy(m0, acc.at[pl.ds(tb + r0 * MB, MB)])

    def load_chunk(cn, par):
        pltpu.async_copy(row_hbm.at[wid, cn], row_r.at[par], isem)
        pltpu.async_copy(col_hbm.at[wid, cn], col_r.at[par], isem)
        pltpu.async_copy(w_hbm.at[wid, cn],
                         w_r.at[pl.ds(par * wchunk, wchunk)], isem)

    def wait_chunk(par):
        pltpu.make_async_copy(row_hbm.at[0, 0], row_r.at[par], isem).wait()
        pltpu.make_async_copy(col_hbm.at[0, 0], col_r.at[par], isem).wait()
        pltpu.make_async_copy(w_hbm.at[0, 0],
                              w_r.at[pl.ds(par * wchunk, wchunk)],
                              isem).wait()

    load_chunk(0, 0)
    wait_chunk(0)
    plsc.subcore_barrier()

    def gather(par, j, r):
        pltpu.async_copy(g_hbm.at[row_r.at[par, j]], bufs[r], gsems[r])

    def gather_wait(r):
        pltpu.make_async_copy(g_hbm.at[row_r.at[0, 0]], bufs[r],
                              gsems[r]).wait()

    def scatter(par, j, r):
        pltpu.async_copy(bufs[r], acc.at[col_r.at[par, j]], ssems[r],
                         add=True)

    def scatter_wait(r):
        pltpu.make_async_copy(bufs[r], acc.at[col_r.at[0, 0]],
                              ssems[r]).wait()

    def scale(par, j, r):
        buf = bufs[r]
        woff = par * wchunk + j * MB

        def sbody(i, carry):
            w16 = w_r[pl.ds(woff + i * 16, 16)]
            for u in range(16):
                wv = w16.at[jnp.full((16,), u, jnp.int32)].get(
                    mode="promise_in_bounds")
                e = i * 16 + u
                for k in range(8):
                    sl = pl.ds(k * 16, 16)
                    buf[e, sl] = buf[e, sl] * wv
            return carry

        lax.fori_loop(0, MB // 16, sbody, 0)

    gather(0, 0, 0)

    def chunk_body(cn, carry):
        par = lax.rem(cn, 2)
        nxt = 1 - par
        for j in range(SB):
            r = j % 2
            gather_wait(r)
            if j == 2:
                @pl.when(cn <= NCH - 2)
                def _():
                    load_chunk(cn + 1, nxt)
            if j >= 1:
                scatter_wait(1 - r)
            else:
                @pl.when(cn >= 1)
                def _():
                    scatter_wait(1 - r)
            if j < SB - 1:
                gather(par, j + 1, 1 - r)
            else:
                @pl.when(cn <= NCH - 2)
                def _():
                    wait_chunk(nxt)
                    gather(nxt, 0, 1 - r)
            scale(par, j, r)
            scatter(par, j, r)
        return carry

    lax.fori_loop(0, NCH, chunk_body, 0)
    scatter_wait((NSUB - 1) % 2)
    plsc.subcore_barrier()
    pltpu.sync_copy(acc.at[pl.ds(tb, ROWS_PER_TILE)],
                    p_hbm.at[pl.ds(tb, ROWS_PER_TILE)])


_BM = 512
_GRID = NPAD // _BM


def _dis(d0_ref, d1_ref):
    deg = d0_ref[...] + d1_ref[...] + 1.0
    return jnp.where(deg > 0, lax.rsqrt(jnp.maximum(deg, 1e-12)), 0.0)


def _dot(a, b):
    return jnp.dot(a, b, preferred_element_type=jnp.float32)


def _enc_body(x_ref, w1, b1, w2, b2, wc, d0, d1, o_ref):
    h = jnp.maximum(_dot(x_ref[...], w1[...]) + b1[...], 0.0)
    h = _dot(h, w2[...]) + b2[...]
    o_ref[...] = _dis(d0, d1) * _dot(h, wc[...])


def _mid_body(pa, pb, g, d0, d1, cb, wn, t_ref, o_ref):
    dis = _dis(d0, d1)
    t = jnp.maximum(dis * (pa[...] + pb[...] + g[...]) + cb[...], 0.0)
    t_ref[...] = t
    o_ref[...] = dis * _dot(t, wn[...])


def _dec_body(t_in, w1, b1, w2, b2, o_ref):
    t = jnp.maximum(_dot(t_in[...], w1[...]) + b1[...], 0.0)
    o = _dot(t, w2[...]) + b2[...]
    o = o - jnp.max(o, axis=1, keepdims=True)
    ex = jnp.exp(o)
    o_ref[...] = ex / jnp.sum(ex, axis=1, keepdims=True)


def _row_spec():
    return pl.BlockSpec((_BM, D), lambda i: (i, 0))


def _col_spec():
    return pl.BlockSpec((_BM, 1), lambda i: (i, 0))


def _wt_spec():
    return pl.BlockSpec((D, D), lambda i: (0, 0))


def _b_spec():
    return pl.BlockSpec((1, D), lambda i: (0, 0))


def _tc_call(body, in_specs, out_dtypes=(jnp.float32,)):
    outs = tuple(jax.ShapeDtypeStruct((NPAD, D), dt) for dt in out_dtypes)
    specs = tuple(_row_spec() for _ in out_dtypes)
    if len(outs) == 1:
        outs, specs = outs[0], specs[0]
    return pl.pallas_call(
        body,
        grid=(_GRID,),
        in_specs=in_specs,
        out_specs=specs,
        out_shape=outs,
    )


def kernel(x, edge_index, edge_weight, enc_w1, enc_b1, enc_w2, enc_b2,
           conv1_w, conv1_b, conv2_w, conv2_b, dec_w1, dec_b1, dec_w2,
           dec_b2):
    def chunk(a, fill):
        a = a.reshape(NW, EW)
        a = jnp.pad(a, ((0, 0), (0, EWP - EW)), constant_values=fill)
        return a.reshape(NW, NBLK, 128)

    def chunk1(a, fill):
        a = a.reshape(NW1, EW1)
        a = jnp.pad(a, ((0, 0), (0, EWP1 - EW1)), constant_values=fill)
        return a.reshape(NW1, NCH, SB, MB)

    rows = chunk(edge_index[0], 0)
    cols = chunk(edge_index[1], 0)
    wgts = chunk(edge_weight, 0.0)
    halves = []
    for h in range(2):
        sl = slice(h * EH, (h + 1) * EH)
        halves.append(
            (chunk1(edge_index[0, sl], 0),
             chunk1(edge_index[1, sl], 0),
             chunk1(edge_weight[sl], 0.0).reshape(NW1, NCH, SB * MB)))
    x_p = jnp.pad(x, ((0, NPAD - N), (0, 0)))

    deg0, deg1 = _deg_kernel(wgts, cols)
    d0 = deg0[:, None]
    d1 = deg1[:, None]

    b1r = enc_b1[None, :]
    b2r = enc_b2[None, :]
    c1b = conv1_b[None, :]
    c2b = conv2_b[None, :]
    db1 = dec_b1[None, :]
    db2 = dec_b2[None, :]

    g1 = _tc_call(_enc_body,
                  [_row_spec(), _wt_spec(), _b_spec(), _wt_spec(),
                   _b_spec(), _wt_spec(), _col_spec(), _col_spec()])(
        x_p, enc_w1, b1r, enc_w2, b2r, conv1_w, d0, d1)

    Ws = jnp.stack([conv2_w, conv2_w])
    bs = jnp.stack([c1b, c2b])

    def conv_step(g, wb):
        wn, cb = wb
        pa = _mp_kernel(g, *halves[0])
        pb = _mp_kernel(g, *halves[1])
        t, gn = _tc_call(_mid_body,
                         [_row_spec(), _row_spec(), _row_spec(),
                          _col_spec(), _col_spec(), _b_spec(), _wt_spec()],
                         out_dtypes=(jnp.float32, jnp.float32))(
            pa, pb, g, d0, d1, cb, wn)
        return gn, t

    _, ts = lax.scan(conv_step, g1, (Ws, bs))

    out = _tc_call(_dec_body,
                   [_row_spec(), _wt_spec(), _b_spec(),
                    _wt_spec(), _b_spec()])(
        ts[1], dec_w1, db1, dec_w2, db2)

    return out[:N]


# speedup vs baseline: 1.0342x; 1.0342x over previous
import functools

import jax
import jax.numpy as jnp
from jax import lax
from jax.experimental import pallas as pl
from jax.experimental.pallas import tpu as pltpu
from jax.experimental.pallas import tpu_sc as plsc

N = 10000
E = 320000
D = 128
NPAD = 10240
NW = 32
EW = E // NW
NBLK = 81
EWP = NBLK * 128
NW1 = 16
EW1 = E // NW1
MB = 128
SB = 8
NCH = 20
EWP1 = NCH * SB * MB
NSUB = NCH * SB
ROWS_PER_TILE = NPAD // 16

_MESH = plsc.VectorSubcoreMesh(core_axis_name="c", subcore_axis_name="s")
_MESH1 = plsc.VectorSubcoreMesh(core_axis_name="c", subcore_axis_name="s",
                                num_cores=1)


@functools.partial(
    pl.kernel,
    out_type=(jax.ShapeDtypeStruct((NPAD,), jnp.float32),
              jax.ShapeDtypeStruct((NPAD,), jnp.float32)),
    mesh=_MESH,
    scratch_types=[
        pltpu.VMEM((NBLK, 128), jnp.float32),
        pltpu.VMEM((NBLK, 128), jnp.int32),
        pltpu.VMEM((ROWS_PER_TILE,), jnp.float32),
        pltpu.VMEM_SHARED((NPAD,), jnp.float32),
    ],
)
def _deg_kernel(w_hbm, col_hbm, deg0_hbm, deg1_hbm, w_v, col_v, zb, acc):
    c = lax.axis_index("c")
    s = lax.axis_index("s")
    wid = s * 2 + c
    tb = s * ROWS_PER_TILE

    for i in range(ROWS_PER_TILE // 16):
        zb[pl.ds(i * 16, 16)] = jnp.zeros((16,), jnp.float32)
    pltpu.sync_copy(zb, acc.at[pl.ds(tb, ROWS_PER_TILE)])
    pltpu.sync_copy(w_hbm.at[wid], w_v)
    pltpu.sync_copy(col_hbm.at[wid], col_v)
    plsc.subcore_barrier()

    def body(j, carry):
        pltpu.sync_copy(w_v.at[j], acc.at[col_v.at[j]], add=True)
        return carry

    lax.fori_loop(0, NBLK, body, 0)
    plsc.subcore_barrier()

    @pl.when(c == 0)
    def _():
        pltpu.sync_copy(acc.at[pl.ds(tb, ROWS_PER_TILE)],
                        deg0_hbm.at[pl.ds(tb, ROWS_PER_TILE)])

    @pl.when(c == 1)
    def _():
        pltpu.sync_copy(acc.at[pl.ds(tb, ROWS_PER_TILE)],
                        deg1_hbm.at[pl.ds(tb, ROWS_PER_TILE)])


@functools.partial(
    pl.kernel,
    out_type=jax.ShapeDtypeStruct((NPAD, D), jnp.float32),
    mesh=_MESH1,
    scratch_types=[
        pltpu.VMEM((2, SB, MB), jnp.int32),
        pltpu.VMEM((2, SB, MB), jnp.int32),
        pltpu.VMEM((2 * SB * MB,), jnp.float32),
        pltpu.VMEM((MB, D), jnp.float32),
        pltpu.VMEM((MB, D), jnp.float32),
        pltpu.VMEM_SHARED((NPAD, D), jnp.float32),
        pltpu.SemaphoreType.DMA,
        pltpu.SemaphoreType.DMA,
        pltpu.SemaphoreType.DMA,
        pltpu.SemaphoreType.DMA,
        pltpu.SemaphoreType.DMA,
    ],
)
def _mp_kernel(g_hbm, row_hbm, col_hbm, w_hbm, p_hbm,
               row_r, col_r, w_r, m0, m1, acc,
               gs0, gs1, ss0, ss1, isem):
    s = lax.axis_index("s")
    wid = s
    tb = s * ROWS_PER_TILE
    bufs = (m0, m1)
    gsems = (gs0, gs1)
    ssems = (ss0, ss1)
    wchunk = SB * MB

    def zbody(r, carry):
        for k in range(8):
            m0[r, pl.ds(k * 16, 16)] = jnp.zeros((16,), jnp.float32)
        return carry

    lax.fori_loop(0, MB, zbody, 0)
    for r0 in range(ROWS_PER_TILE // MB):
        pltpu.sync_copy(m0, acc.at[pl.ds(tb + r0 * MB, MB)])

    def load_chunk(cn, par):
        pltpu.async_copy(row_hbm.at[wid, cn], row_r.at[par], isem)
        pltpu.async_copy(col_hbm.at[wid, cn], col_r.at[par], isem)
        pltpu.async_copy(w_hbm.at[wid, cn],
                         w_r.at[pl.ds(par * wchunk, wchunk)], isem)

    def wait_chunk(par):
        pltpu.make_async_copy(row_hbm.at[0, 0], row_r.at[par], isem).wait()
        pltpu.make_async_copy(col_hbm.at[0, 0], col_r.at[par], isem).wait()
        pltpu.make_async_copy(w_hbm.at[0, 0],
                              w_r.at[pl.ds(par * wchunk, wchunk)],
                              isem).wait()

    load_chunk(0, 0)
    wait_chunk(0)
    plsc.subcore_barrier()

    def gather(par, j, r):
        pltpu.async_copy(g_hbm.at[row_r.at[par, j]], bufs[r], gsems[r])

    def gather_wait(r):
        pltpu.make_async_copy(g_hbm.at[row_r.at[0, 0]], bufs[r],
                              gsems[r]).wait()

    def scatter(par, j, r):
        pltpu.async_copy(bufs[r], acc.at[col_r.at[par, j]], ssems[r],
                         add=True)

    def scatter_wait(r):
        pltpu.make_async_copy(bufs[r], acc.at[col_r.at[0, 0]],
                              ssems[r]).wait()

    def scale(par, j, r):
        buf = bufs[r]
        woff = par * wchunk + j * MB

        def sbody(i, carry):
            w16 = w_r[pl.ds(woff + i * 16, 16)]
            for u in range(16):
                wv = w16.at[jnp.full((16,), u, jnp.int32)].get(
                    mode="promise_in_bounds")
                e = i * 16 + u
                for k in range(8):
                    sl = pl.ds(k * 16, 16)
                    buf[e, sl] = buf[e, sl] * wv
            return carry

        lax.fori_loop(0, MB // 16, sbody, 0)

    gather(0, 0, 0)

    def chunk_body(cn, carry):
        par = lax.rem(cn, 2)
        nxt = 1 - par
        for j in range(SB):
            r = j % 2
            gather_wait(r)
            if j == 2:
                @pl.when(cn <= NCH - 2)
                def _():
                    load_chunk(cn + 1, nxt)
            if j >= 1:
                scatter_wait(1 - r)
            else:
                @pl.when(cn >= 1)
                def _():
                    scatter_wait(1 - r)
            if j < SB - 1:
                gather(par, j + 1, 1 - r)
            else:
                @pl.when(cn <= NCH - 2)
                def _():
                    wait_chunk(nxt)
                    gather(nxt, 0, 1 - r)
            scale(par, j, r)
            scatter(par, j, r)
        return carry

    lax.fori_loop(0, NCH, chunk_body, 0)
    scatter_wait((NSUB - 1) % 2)
    plsc.subcore_barrier()
    pltpu.sync_copy(acc.at[pl.ds(tb, ROWS_PER_TILE)],
                    p_hbm.at[pl.ds(tb, ROWS_PER_TILE)])


_BM = 512
_GRID = NPAD // _BM


def _dis(d0_ref, d1_ref):
    deg = d0_ref[...] + d1_ref[...] + 1.0
    return jnp.where(deg > 0, lax.rsqrt(jnp.maximum(deg, 1e-12)), 0.0)


def _dot(a, b):
    return jnp.dot(a, b, preferred_element_type=jnp.float32)


def _enc_body(x_ref, w1, b1, w2, b2, wc, d0, d1, o_ref):
    h = jnp.maximum(_dot(x_ref[...], w1[...]) + b1[...], 0.0)
    h = _dot(h, w2[...]) + b2[...]
    o_ref[...] = _dis(d0, d1) * _dot(h, wc[...])


def _mid_body(p, g, d0, d1, cb, wn, t_ref, o_ref):
    dis = _dis(d0, d1)
    t = jnp.maximum(dis * (p[...] + g[...]) + cb[...], 0.0)
    t_ref[...] = t
    o_ref[...] = dis * _dot(t, wn[...])


def _dec_body(t_in, w1, b1, w2, b2, o_ref):
    t = jnp.maximum(_dot(t_in[...], w1[...]) + b1[...], 0.0)
    o = _dot(t, w2[...]) + b2[...]
    o = o - jnp.max(o, axis=1, keepdims=True)
    ex = jnp.exp(o)
    o_ref[...] = ex / jnp.sum(ex, axis=1, keepdims=True)


def _row_spec():
    return pl.BlockSpec((_BM, D), lambda i: (i, 0))


def _col_spec():
    return pl.BlockSpec((_BM, 1), lambda i: (i, 0))


def _wt_spec():
    return pl.BlockSpec((D, D), lambda i: (0, 0))


def _b_spec():
    return pl.BlockSpec((1, D), lambda i: (0, 0))


def _tc_call(body, in_specs, out_dtypes=(jnp.float32,)):
    outs = tuple(jax.ShapeDtypeStruct((NPAD, D), dt) for dt in out_dtypes)
    specs = tuple(_row_spec() for _ in out_dtypes)
    if len(outs) == 1:
        outs, specs = outs[0], specs[0]
    return pl.pallas_call(
        body,
        grid=(_GRID,),
        in_specs=in_specs,
        out_specs=specs,
        out_shape=outs,
    )


def kernel(x, edge_index, edge_weight, enc_w1, enc_b1, enc_w2, enc_b2,
           conv1_w, conv1_b, conv2_w, conv2_b, dec_w1, dec_b1, dec_w2,
           dec_b2):
    def chunk(a, fill):
        a = a.reshape(NW, EW)
        a = jnp.pad(a, ((0, 0), (0, EWP - EW)), constant_values=fill)
        return a.reshape(NW, NBLK, 128)

    def chunk1(a, fill):
        a = a.reshape(NW1, EW1)
        a = jnp.pad(a, ((0, 0), (0, EWP1 - EW1)), constant_values=fill)
        return a.reshape(NW1, NCH, SB, MB)

    rows = chunk(edge_index[0], 0)
    cols = chunk(edge_index[1], 0)
    wgts = chunk(edge_weight, 0.0)
    rows1 = chunk1(edge_index[0], 0)
    cols1 = chunk1(edge_index[1], 0)
    wgts1 = chunk1(edge_weight, 0.0).reshape(NW1, NCH, SB * MB)
    x_p = jnp.pad(x, ((0, NPAD - N), (0, 0)))

    deg0, deg1 = _deg_kernel(wgts, cols)
    d0 = deg0[:, None]
    d1 = deg1[:, None]

    b1r = enc_b1[None, :]
    b2r = enc_b2[None, :]
    c1b = conv1_b[None, :]
    c2b = conv2_b[None, :]
    db1 = dec_b1[None, :]
    db2 = dec_b2[None, :]

    g1 = _tc_call(_enc_body,
                  [_row_spec(), _wt_spec(), _b_spec(), _wt_spec(),
                   _b_spec(), _wt_spec(), _col_spec(), _col_spec()])(
        x_p, enc_w1, b1r, enc_w2, b2r, conv1_w, d0, d1)

    Ws = jnp.stack([conv2_w, conv2_w])
    bs = jnp.stack([c1b, c2b])

    def conv_step(g, wb):
        wn, cb = wb
        p = _mp_kernel(g, rows1, cols1, wgts1)
        t, gn = _tc_call(_mid_body,
                         [_row_spec(), _row_spec(), _col_spec(),
                          _col_spec(), _b_spec(), _wt_spec()],
                         out_dtypes=(jnp.float32, jnp.float32))(
            p, g, d0, d1, cb, wn)
        return gn, t

    _, ts = lax.scan(conv_step, g1, (Ws, bs))

    out = _tc_call(_dec_body,
                   [_row_spec(), _wt_spec(), _b_spec(),
                    _wt_spec(), _b_spec()])(
        ts[1], dec_w1, db1, dec_w2, db2)

    return out[:N]


# speedup vs baseline: 1.0472x; 1.0126x over previous
import functools

import jax
import jax.numpy as jnp
from jax import lax
from jax.experimental import pallas as pl
from jax.experimental.pallas import tpu as pltpu
from jax.experimental.pallas import tpu_sc as plsc

N = 10000
E = 320000
D = 128
NPAD = 10240
NW = 32
EW = E // NW
NBLK = 81
EWP = NBLK * 128
NW1 = 16
EW1 = E // NW1
MB = 128
SB = 8
NCH = 20
EWP1 = NCH * SB * MB
NSUB = NCH * SB
ROWS_PER_TILE = NPAD // 16

_MESH = plsc.VectorSubcoreMesh(core_axis_name="c", subcore_axis_name="s")
_MESH1 = plsc.VectorSubcoreMesh(core_axis_name="c", subcore_axis_name="s",
                                num_cores=1)


@functools.partial(
    pl.kernel,
    out_type=(jax.ShapeDtypeStruct((NPAD,), jnp.float32),
              jax.ShapeDtypeStruct((NPAD,), jnp.float32)),
    mesh=_MESH,
    scratch_types=[
        pltpu.VMEM((NBLK, 128), jnp.float32),
        pltpu.VMEM((NBLK, 128), jnp.int32),
        pltpu.VMEM((ROWS_PER_TILE,), jnp.float32),
        pltpu.VMEM_SHARED((NPAD,), jnp.float32),
    ],
)
def _deg_kernel(w_hbm, col_hbm, deg0_hbm, deg1_hbm, w_v, col_v, zb, acc):
    c = lax.axis_index("c")
    s = lax.axis_index("s")
    wid = s * 2 + c
    tb = s * ROWS_PER_TILE

    for i in range(ROWS_PER_TILE // 16):
        zb[pl.ds(i * 16, 16)] = jnp.zeros((16,), jnp.float32)
    pltpu.sync_copy(zb, acc.at[pl.ds(tb, ROWS_PER_TILE)])
    pltpu.sync_copy(w_hbm.at[wid], w_v)
    pltpu.sync_copy(col_hbm.at[wid], col_v)
    plsc.subcore_barrier()

    def body(j, carry):
        pltpu.sync_copy(w_v.at[j], acc.at[col_v.at[j]], add=True)
        return carry

    lax.fori_loop(0, NBLK, body, 0)
    plsc.subcore_barrier()

    @pl.when(c == 0)
    def _():
        pltpu.sync_copy(acc.at[pl.ds(tb, ROWS_PER_TILE)],
                        deg0_hbm.at[pl.ds(tb, ROWS_PER_TILE)])

    @pl.when(c == 1)
    def _():
        pltpu.sync_copy(acc.at[pl.ds(tb, ROWS_PER_TILE)],
                        deg1_hbm.at[pl.ds(tb, ROWS_PER_TILE)])


@functools.partial(
    pl.kernel,
    out_type=jax.ShapeDtypeStruct((NPAD, D), jnp.float32),
    mesh=_MESH1,
    scratch_types=[
        pltpu.VMEM((2, SB, MB), jnp.int32),
        pltpu.VMEM((2, SB, MB), jnp.int32),
        pltpu.VMEM((2 * SB * MB,), jnp.float32),
        pltpu.VMEM((MB, D), jnp.float32),
        pltpu.VMEM((MB, D), jnp.float32),
        pltpu.VMEM_SHARED((NPAD, D), jnp.float32),
        pltpu.SemaphoreType.DMA,
        pltpu.SemaphoreType.DMA,
        pltpu.SemaphoreType.DMA,
        pltpu.SemaphoreType.DMA,
        pltpu.SemaphoreType.DMA,
    ],
)
def _mp_kernel(g_hbm, row_hbm, col_hbm, w_hbm, p_hbm,
               row_r, col_r, w_r, m0, m1, acc,
               gs0, gs1, ss0, ss1, isem):
    s = lax.axis_index("s")
    wid = s
    tb = s * ROWS_PER_TILE
    bufs = (m0, m1)
    gsems = (gs0, gs1)
    ssems = (ss0, ss1)
    wchunk = SB * MB

    def zbody(r, carry):
        for k in range(8):
            m0[r, pl.ds(k * 16, 16)] = jnp.zeros((16,), jnp.float32)
        return carry

    lax.fori_loop(0, MB, zbody, 0)
    for r0 in range(ROWS_PER_TILE // MB):
        pltpu.sync_copy(m0, acc.at[pl.ds(tb + r0 * MB, MB)])

    def load_chunk(cn, par):
        pltpu.async_copy(row_hbm.at[wid, cn], row_r.at[par], isem)
        pltpu.async_copy(col_hbm.at[wid, cn], col_r.at[par], isem)
        pltpu.async_copy(w_hbm.at[wid, cn],
                         w_r.at[pl.ds(par * wchunk, wchunk)], isem)

    def wait_chunk(par):
        pltpu.make_async_copy(row_hbm.at[0, 0], row_r.at[par], isem).wait()
        pltpu.make_async_copy(col_hbm.at[0, 0], col_r.at[par], isem).wait()
        pltpu.make_async_copy(w_hbm.at[0, 0],
                              w_r.at[pl.ds(par * wchunk, wchunk)],
                              isem).wait()

    load_chunk(0, 0)
    wait_chunk(0)
    plsc.subcore_barrier()

    def gather(par, j, r):
        pltpu.async_copy(g_hbm.at[row_r.at[par, j]], bufs[r], gsems[r])

    def gather_wait(r):
        pltpu.make_async_copy(g_hbm.at[row_r.at[0, 0]], bufs[r],
                              gsems[r]).wait()

    def scatter(par, j, r):
        pltpu.async_copy(bufs[r], acc.at[col_r.at[par, j]], ssems[r],
                         add=True)

    def scatter_wait(r):
        pltpu.make_async_copy(bufs[r], acc.at[col_r.at[0, 0]],
                              ssems[r]).wait()

    def scale(par, j, r):
        buf = bufs[r]
        woff = par * wchunk + j * MB

        def sbody(i, carry):
            w16 = w_r[pl.ds(woff + i * 16, 16)]
            for u in range(16):
                wv = w16.at[jnp.full((16,), u, jnp.int32)].get(
                    mode="promise_in_bounds")
                e = i * 16 + u
                for k in range(8):
                    sl = pl.ds(k * 16, 16)
                    buf[e, sl] = buf[e, sl] * wv
            return carry

        lax.fori_loop(0, MB // 16, sbody, 0)

    gather(0, 0, 0)

    def chunk_body(cn, carry):
        par = lax.rem(cn, 2)
        nxt = 1 - par
        for j in range(SB):
            r = j % 2
            gather_wait(r)
            if j == 2:
                @pl.when(cn <= NCH - 2)
                def _():
                    load_chunk(cn + 1, nxt)
            if j >= 1:
                scatter_wait(1 - r)
            else:
                @pl.when(cn >= 1)
                def _():
                    scatter_wait(1 - r)
            if j < SB - 1:
                gather(par, j + 1, 1 - r)
            else:
                @pl.when(cn <= NCH - 2)
                def _():
                    wait_chunk(nxt)
                    gather(nxt, 0, 1 - r)
            scale(par, j, r)
            scatter(par, j, r)
        return carry

    lax.fori_loop(0, NCH, chunk_body, 0)
    scatter_wait((NSUB - 1) % 2)
    plsc.subcore_barrier()
    pltpu.sync_copy(acc.at[pl.ds(tb, ROWS_PER_TILE)],
                    p_hbm.at[pl.ds(tb, ROWS_PER_TILE)])


_BM = 512
_GRID = NPAD // _BM


def _dis(d0_ref, d1_ref):
    deg = d0_ref[...] + d1_ref[...] + 1.0
    return jnp.where(deg > 0, lax.rsqrt(jnp.maximum(deg, 1e-12)), 0.0)


def _dot(a, b):
    return jnp.dot(a, b, preferred_element_type=jnp.float32)


def _enc_body(x_ref, w1, b1, w2, b2, wc, d0, d1, o_ref):
    h = jnp.maximum(_dot(x_ref[...], w1[...]) + b1[...], 0.0)
    h = _dot(h, w2[...]) + b2[...]
    o_ref[...] = _dis(d0, d1) * _dot(h, wc[...])


def _mid_body(p, g, d0, d1, cb, wn, t_ref, o_ref):
    dis = _dis(d0, d1)
    t = jnp.maximum(dis * (p[...] + g[...]) + cb[...], 0.0)
    t_ref[...] = t
    o_ref[...] = dis * _dot(t, wn[...])


def _dec_body(t_in, w1, b1, w2, b2, o_ref):
    t = jnp.maximum(_dot(t_in[...], w1[...]) + b1[...], 0.0)
    o = _dot(t, w2[...]) + b2[...]
    o = o - jnp.max(o, axis=1, keepdims=True)
    ex = jnp.exp(o)
    o_ref[...] = ex / jnp.sum(ex, axis=1, keepdims=True)


def _row_spec():
    return pl.BlockSpec((_BM, D), lambda i: (i, 0))


def _col_spec():
    return pl.BlockSpec((_BM, 1), lambda i: (i, 0))


def _wt_spec():
    return pl.BlockSpec((D, D), lambda i: (0, 0))


def _b_spec():
    return pl.BlockSpec((1, D), lambda i: (0, 0))


def _tc_call(body, in_specs, out_dtypes=(jnp.float32,)):
    outs = tuple(jax.ShapeDtypeStruct((NPAD, D), dt) for dt in out_dtypes)
    specs = tuple(_row_spec() for _ in out_dtypes)
    if len(outs) == 1:
        outs, specs = outs[0], specs[0]
    return pl.pallas_call(
        body,
        grid=(_GRID,),
        in_specs=in_specs,
        out_specs=specs,
        out_shape=outs,
    )


def kernel(x, edge_index, edge_weight, enc_w1, enc_b1, enc_w2, enc_b2,
           conv1_w, conv1_b, conv2_w, conv2_b, dec_w1, dec_b1, dec_w2,
           dec_b2):
    def chunk(a, fill):
        a = a.reshape(NW, EW)
        a = jnp.pad(a, ((0, 0), (0, EWP - EW)), constant_values=fill)
        return a.reshape(NW, NBLK, 128)

    def chunk1(a, fill):
        a = a.reshape(NW1, EW1)
        a = jnp.pad(a, ((0, 0), (0, EWP1 - EW1)), constant_values=fill)
        return a.reshape(NW1, NCH, SB, MB)

    rows = chunk(edge_index[0], 0)
    cols = chunk(edge_index[1], 0)
    wgts = chunk(edge_weight, 0.0)
    rows1 = chunk1(edge_index[0], 0)
    cols1 = chunk1(edge_index[1], 0)
    wgts1 = chunk1(edge_weight, 0.0).reshape(NW1, NCH, SB * MB)
    x_p = jnp.pad(x, ((0, NPAD - N), (0, 0)))

    deg0, deg1 = _deg_kernel(wgts, cols)
    d0 = deg0[:, None]
    d1 = deg1[:, None]

    b1r = enc_b1[None, :]
    b2r = enc_b2[None, :]
    c1b = conv1_b[None, :]
    c2b = conv2_b[None, :]
    db1 = dec_b1[None, :]
    db2 = dec_b2[None, :]

    g1 = _tc_call(_enc_body,
                  [_row_spec(), _wt_spec(), _b_spec(), _wt_spec(),
                   _b_spec(), _wt_spec(), _col_spec(), _col_spec()])(
        x_p, enc_w1, b1r, enc_w2, b2r, conv1_w, d0, d1)

    def conv_step(g, wn, cb):
        p = _mp_kernel(g, rows1, cols1, wgts1)
        return _tc_call(_mid_body,
                        [_row_spec(), _row_spec(), _col_spec(),
                         _col_spec(), _b_spec(), _wt_spec()],
                        out_dtypes=(jnp.float32, jnp.float32))(
            p, g, d0, d1, cb, wn)

    _, g2 = conv_step(g1, conv2_w, c1b)
    t2, _ = conv_step(g2, conv2_w, c2b)

    out = _tc_call(_dec_body,
                   [_row_spec(), _wt_spec(), _b_spec(),
                    _wt_spec(), _b_spec()])(
        t2, dec_w1, db1, dec_w2, db2)

    return out[:N]


# speedup vs baseline: 1.0623x; 1.0144x over previous
import functools

import jax
import jax.numpy as jnp
from jax import lax
from jax.experimental import pallas as pl
from jax.experimental.pallas import tpu as pltpu
from jax.experimental.pallas import tpu_sc as plsc

N = 10000
E = 320000
D = 128
NPAD = 10240
NW = 32
EW = E // NW
NBLK = 81
EWP = NBLK * 128
NW1 = 16
EW1 = E // NW1
MB = 128
SB = 8
NCH = 20
EWP1 = NCH * SB * MB
NSUB = NCH * SB
ROWS_PER_TILE = NPAD // 16

_MESH = plsc.VectorSubcoreMesh(core_axis_name="c", subcore_axis_name="s")
_MESH1 = plsc.VectorSubcoreMesh(core_axis_name="c", subcore_axis_name="s",
                                num_cores=1)


@functools.partial(
    pl.kernel,
    out_type=(jax.ShapeDtypeStruct((NPAD,), jnp.float32),
              jax.ShapeDtypeStruct((NPAD,), jnp.float32)),
    mesh=_MESH,
    scratch_types=[
        pltpu.VMEM((NBLK, 128), jnp.float32),
        pltpu.VMEM((NBLK, 128), jnp.int32),
        pltpu.VMEM((ROWS_PER_TILE,), jnp.float32),
        pltpu.VMEM_SHARED((NPAD,), jnp.float32),
    ],
)
def _deg_kernel(w_hbm, col_hbm, deg0_hbm, deg1_hbm, w_v, col_v, zb, acc):
    c = lax.axis_index("c")
    s = lax.axis_index("s")
    wid = s * 2 + c
    tb = s * ROWS_PER_TILE

    for i in range(ROWS_PER_TILE // 16):
        zb[pl.ds(i * 16, 16)] = jnp.zeros((16,), jnp.float32)
    pltpu.sync_copy(zb, acc.at[pl.ds(tb, ROWS_PER_TILE)])
    pltpu.sync_copy(w_hbm.at[wid], w_v)
    pltpu.sync_copy(col_hbm.at[wid], col_v)
    plsc.subcore_barrier()

    def body(j, carry):
        pltpu.sync_copy(w_v.at[j], acc.at[col_v.at[j]], add=True)
        return carry

    lax.fori_loop(0, NBLK, body, 0)
    plsc.subcore_barrier()

    @pl.when(c == 0)
    def _():
        pltpu.sync_copy(acc.at[pl.ds(tb, ROWS_PER_TILE)],
                        deg0_hbm.at[pl.ds(tb, ROWS_PER_TILE)])

    @pl.when(c == 1)
    def _():
        pltpu.sync_copy(acc.at[pl.ds(tb, ROWS_PER_TILE)],
                        deg1_hbm.at[pl.ds(tb, ROWS_PER_TILE)])


@functools.partial(
    pl.kernel,
    out_type=jax.ShapeDtypeStruct((NPAD, D), jnp.float32),
    mesh=_MESH1,
    scratch_types=[
        pltpu.VMEM((2, SB, MB), jnp.int32),
        pltpu.VMEM((2, SB, MB), jnp.int32),
        pltpu.VMEM((2 * SB * MB,), jnp.float32),
        pltpu.VMEM((MB, D), jnp.float32),
        pltpu.VMEM((MB, D), jnp.float32),
        pltpu.VMEM_SHARED((NPAD, D), jnp.float32),
        pltpu.SemaphoreType.DMA,
        pltpu.SemaphoreType.DMA,
        pltpu.SemaphoreType.DMA,
        pltpu.SemaphoreType.DMA,
        pltpu.SemaphoreType.DMA,
    ],
)
def _mp_kernel(g_hbm, row_hbm, col_hbm, w_hbm, p_hbm,
               row_r, col_r, w_r, m0, m1, acc,
               gs0, gs1, ss0, ss1, isem):
    s = lax.axis_index("s")
    wid = s
    tb = s * ROWS_PER_TILE
    bufs = (m0, m1)
    gsems = (gs0, gs1)
    ssems = (ss0, ss1)
    wchunk = SB * MB

    def zbody(r, carry):
        for k in range(8):
            m0[r, pl.ds(k * 16, 16)] = jnp.zeros((16,), jnp.float32)
        return carry

    lax.fori_loop(0, MB, zbody, 0)
    for r0 in range(ROWS_PER_TILE // MB):
        pltpu.sync_copy(m0, acc.at[pl.ds(tb + r0 * MB, MB)])

    def load_chunk(cn, par):
        pltpu.async_copy(row_hbm.at[wid, cn], row_r.at[par], isem)
        pltpu.async_copy(col_hbm.at[wid, cn], col_r.at[par], isem)
        pltpu.async_copy(w_hbm.at[wid, cn],
                         w_r.at[pl.ds(par * wchunk, wchunk)], isem)

    def wait_chunk(par):
        pltpu.make_async_copy(row_hbm.at[0, 0], row_r.at[par], isem).wait()
        pltpu.make_async_copy(col_hbm.at[0, 0], col_r.at[par], isem).wait()
        pltpu.make_async_copy(w_hbm.at[0, 0],
                              w_r.at[pl.ds(par * wchunk, wchunk)],
                              isem).wait()

    load_chunk(0, 0)
    wait_chunk(0)
    plsc.subcore_barrier()

    def gather(par, j, r):
        pltpu.async_copy(g_hbm.at[row_r.at[par, j]], bufs[r], gsems[r])

    def gather_wait(r):
        pltpu.make_async_copy(g_hbm.at[row_r.at[0, 0]], bufs[r],
                              gsems[r]).wait()

    def scatter(par, j, r):
        pltpu.async_copy(bufs[r], acc.at[col_r.at[par, j]], ssems[r],
                         add=True)

    def scatter_wait(r):
        pltpu.make_async_copy(bufs[r], acc.at[col_r.at[0, 0]],
                              ssems[r]).wait()

    def scale(par, j, r):
        buf = bufs[r]
        woff = par * wchunk + j * MB

        def sbody(i, carry):
            w16 = w_r[pl.ds(woff + i * 16, 16)]
            for u in range(16):
                wv = w16.at[jnp.full((16,), u, jnp.int32)].get(
                    mode="promise_in_bounds")
                e = i * 16 + u
                for k in range(8):
                    sl = pl.ds(k * 16, 16)
                    buf[e, sl] = buf[e, sl] * wv
            return carry

        lax.fori_loop(0, MB // 16, sbody, 0)

    gather(0, 0, 0)

    def chunk_body(cn, carry):
        par = lax.rem(cn, 2)
        nxt = 1 - par
        for j in range(SB):
            r = j % 2
            gather_wait(r)
            if j == 2:
                @pl.when(cn <= NCH - 2)
                def _():
                    load_chunk(cn + 1, nxt)
            if j >= 1:
                scatter_wait(1 - r)
            else:
                @pl.when(cn >= 1)
                def _():
                    scatter_wait(1 - r)
            if j < SB - 1:
                gather(par, j + 1, 1 - r)
            else:
                @pl.when(cn <= NCH - 2)
                def _():
                    wait_chunk(nxt)
                    gather(nxt, 0, 1 - r)
            scale(par, j, r)
            scatter(par, j, r)
        return carry

    lax.fori_loop(0, NCH, chunk_body, 0)
    scatter_wait((NSUB - 1) % 2)
    plsc.subcore_barrier()
    pltpu.sync_copy(acc.at[pl.ds(tb, ROWS_PER_TILE)],
                    p_hbm.at[pl.ds(tb, ROWS_PER_TILE)])


_BM = 512
_GRID = NPAD // _BM


def _dis(d0_ref, d1_ref):
    deg = d0_ref[...] + d1_ref[...] + 1.0
    return jnp.where(deg > 0, lax.rsqrt(jnp.maximum(deg, 1e-12)), 0.0)


def _dot(a, b):
    return jnp.dot(a, b, preferred_element_type=jnp.float32)


def _enc_body(x_ref, w1, b1, w2, b2, wc, d0, d1, o_ref):
    h = jnp.maximum(_dot(x_ref[...], w1[...]) + b1[...], 0.0)
    h = _dot(h, w2[...]) + b2[...]
    o_ref[...] = _dis(d0, d1) * _dot(h, wc[...])


def _mid_body(p, g, d0, d1, cb, wn, t_ref, o_ref):
    dis = _dis(d0, d1)
    t = jnp.maximum(dis * (p[...] + g[...]) + cb[...], 0.0)
    t_ref[...] = t
    o_ref[...] = dis * _dot(t, wn[...])


def _dec_body(p, g, d0, d1, cb, w1, b1, w2, b2, o_ref):
    dis = _dis(d0, d1)
    t = jnp.maximum(dis * (p[...] + g[...]) + cb[...], 0.0)
    t = jnp.maximum(_dot(t, w1[...]) + b1[...], 0.0)
    o = _dot(t, w2[...]) + b2[...]
    o = o - jnp.max(o, axis=1, keepdims=True)
    ex = jnp.exp(o)
    o_ref[...] = ex / jnp.sum(ex, axis=1, keepdims=True)


def _row_spec():
    return pl.BlockSpec((_BM, D), lambda i: (i, 0))


def _col_spec():
    return pl.BlockSpec((_BM, 1), lambda i: (i, 0))


def _wt_spec():
    return pl.BlockSpec((D, D), lambda i: (0, 0))


def _b_spec():
    return pl.BlockSpec((1, D), lambda i: (0, 0))


def _tc_call(body, in_specs, out_dtypes=(jnp.float32,)):
    outs = tuple(jax.ShapeDtypeStruct((NPAD, D), dt) for dt in out_dtypes)
    specs = tuple(_row_spec() for _ in out_dtypes)
    if len(outs) == 1:
        outs, specs = outs[0], specs[0]
    return pl.pallas_call(
        body,
        grid=(_GRID,),
        in_specs=in_specs,
        out_specs=specs,
        out_shape=outs,
    )


def kernel(x, edge_index, edge_weight, enc_w1, enc_b1, enc_w2, enc_b2,
           conv1_w, conv1_b, conv2_w, conv2_b, dec_w1, dec_b1, dec_w2,
           dec_b2):
    def chunk(a, fill):
        a = a.reshape(NW, EW)
        a = jnp.pad(a, ((0, 0), (0, EWP - EW)), constant_values=fill)
        return a.reshape(NW, NBLK, 128)

    def chunk1(a, fill):
        a = a.reshape(NW1, EW1)
        a = jnp.pad(a, ((0, 0), (0, EWP1 - EW1)), constant_values=fill)
        return a.reshape(NW1, NCH, SB, MB)

    rows = chunk(edge_index[0], 0)
    cols = chunk(edge_index[1], 0)
    wgts = chunk(edge_weight, 0.0)
    rows1 = chunk1(edge_index[0], 0)
    cols1 = chunk1(edge_index[1], 0)
    wgts1 = chunk1(edge_weight, 0.0).reshape(NW1, NCH, SB * MB)
    x_p = jnp.pad(x, ((0, NPAD - N), (0, 0)))

    deg0, deg1 = _deg_kernel(wgts, cols)
    d0 = deg0[:, None]
    d1 = deg1[:, None]

    b1r = enc_b1[None, :]
    b2r = enc_b2[None, :]
    c1b = conv1_b[None, :]
    c2b = conv2_b[None, :]
    db1 = dec_b1[None, :]
    db2 = dec_b2[None, :]

    g1 = _tc_call(_enc_body,
                  [_row_spec(), _wt_spec(), _b_spec(), _wt_spec(),
                   _b_spec(), _wt_spec(), _col_spec(), _col_spec()])(
        x_p, enc_w1, b1r, enc_w2, b2r, conv1_w, d0, d1)

    def conv_step(g, wn, cb):
        p = _mp_kernel(g, rows1, cols1, wgts1)
        return _tc_call(_mid_body,
                        [_row_spec(), _row_spec(), _col_spec(),
                         _col_spec(), _b_spec(), _wt_spec()],
                        out_dtypes=(jnp.float32, jnp.float32))(
            p, g, d0, d1, cb, wn)

    _, g2 = conv_step(g1, conv2_w, c1b)
    p2 = _mp_kernel(g2, rows1, cols1, wgts1)

    out = _tc_call(_dec_body,
                   [_row_spec(), _row_spec(), _col_spec(), _col_spec(),
                    _b_spec(), _wt_spec(), _b_spec(),
                    _wt_spec(), _b_spec()])(
        p2, g2, d0, d1, c2b, dec_w1, db1, dec_w2, db2)

    return out[:N]


# speedup vs baseline: 1.0642x; 1.0018x over previous
import functools

import jax
import jax.numpy as jnp
from jax import lax
from jax.experimental import pallas as pl
from jax.experimental.pallas import tpu as pltpu
from jax.experimental.pallas import tpu_sc as plsc

N = 10000
E = 320000
D = 128
NPAD = 10240
NW = 32
EW = E // NW
NBLK = 81
EWP = NBLK * 128
NW1 = 16
EW1 = E // NW1
MB = 128
SB = 8
NCH = 20
EWP1 = NCH * SB * MB
NSUB = NCH * SB
ROWS_PER_TILE = NPAD // 16

_MESH = plsc.VectorSubcoreMesh(core_axis_name="c", subcore_axis_name="s")
_MESH1 = plsc.VectorSubcoreMesh(core_axis_name="c", subcore_axis_name="s",
                                num_cores=1)


@functools.partial(
    pl.kernel,
    out_type=(jax.ShapeDtypeStruct((NPAD,), jnp.float32),
              jax.ShapeDtypeStruct((NPAD,), jnp.float32)),
    mesh=_MESH,
    scratch_types=[
        pltpu.VMEM((NBLK, 128), jnp.float32),
        pltpu.VMEM((NBLK, 128), jnp.int32),
        pltpu.VMEM((ROWS_PER_TILE,), jnp.float32),
        pltpu.VMEM_SHARED((NPAD,), jnp.float32),
    ],
)
def _deg_kernel(w_hbm, col_hbm, deg0_hbm, deg1_hbm, w_v, col_v, zb, acc):
    c = lax.axis_index("c")
    s = lax.axis_index("s")
    wid = s * 2 + c
    tb = s * ROWS_PER_TILE

    for i in range(ROWS_PER_TILE // 16):
        zb[pl.ds(i * 16, 16)] = jnp.zeros((16,), jnp.float32)
    pltpu.sync_copy(zb, acc.at[pl.ds(tb, ROWS_PER_TILE)])
    pltpu.sync_copy(w_hbm.at[wid], w_v)
    pltpu.sync_copy(col_hbm.at[wid], col_v)
    plsc.subcore_barrier()

    def body(j, carry):
        pltpu.sync_copy(w_v.at[j], acc.at[col_v.at[j]], add=True)
        return carry

    lax.fori_loop(0, NBLK, body, 0)
    plsc.subcore_barrier()

    @pl.when(c == 0)
    def _():
        pltpu.sync_copy(acc.at[pl.ds(tb, ROWS_PER_TILE)],
                        deg0_hbm.at[pl.ds(tb, ROWS_PER_TILE)])

    @pl.when(c == 1)
    def _():
        pltpu.sync_copy(acc.at[pl.ds(tb, ROWS_PER_TILE)],
                        deg1_hbm.at[pl.ds(tb, ROWS_PER_TILE)])


@functools.partial(
    pl.kernel,
    out_type=jax.ShapeDtypeStruct((NPAD, D), jnp.float32),
    mesh=_MESH1,
    scratch_types=[
        pltpu.VMEM((2, SB, MB), jnp.int32),
        pltpu.VMEM((2, SB, MB), jnp.int32),
        pltpu.VMEM((2 * SB * MB,), jnp.float32),
        pltpu.VMEM((MB, D), jnp.float32),
        pltpu.VMEM((MB, D), jnp.float32),
        pltpu.VMEM_SHARED((NPAD, D), jnp.float32),
        pltpu.SemaphoreType.DMA,
        pltpu.SemaphoreType.DMA,
        pltpu.SemaphoreType.DMA,
        pltpu.SemaphoreType.DMA,
        pltpu.SemaphoreType.DMA,
    ],
)
def _mp_kernel(g_hbm, row_hbm, col_hbm, w_hbm, p_hbm,
               row_r, col_r, w_r, m0, m1, acc,
               gs0, gs1, ss0, ss1, isem):
    s = lax.axis_index("s")
    wid = s
    tb = s * ROWS_PER_TILE
    bufs = (m0, m1)
    gsems = (gs0, gs1)
    ssems = (ss0, ss1)
    wchunk = SB * MB

    def zbody(r, carry):
        for k in range(8):
            m0[r, pl.ds(k * 16, 16)] = jnp.zeros((16,), jnp.float32)
        return carry

    lax.fori_loop(0, MB, zbody, 0)
    for r0 in range(ROWS_PER_TILE // MB):
        pltpu.sync_copy(m0, acc.at[pl.ds(tb + r0 * MB, MB)])

    def load_chunk(cn, par):
        pltpu.async_copy(row_hbm.at[wid, cn], row_r.at[par], isem)
        pltpu.async_copy(col_hbm.at[wid, cn], col_r.at[par], isem)
        pltpu.async_copy(w_hbm.at[wid, cn],
                         w_r.at[pl.ds(par * wchunk, wchunk)], isem)

    def wait_chunk(par):
        pltpu.make_async_copy(row_hbm.at[0, 0], row_r.at[par], isem).wait()
        pltpu.make_async_copy(col_hbm.at[0, 0], col_r.at[par], isem).wait()
        pltpu.make_async_copy(w_hbm.at[0, 0],
                              w_r.at[pl.ds(par * wchunk, wchunk)],
                              isem).wait()

    load_chunk(0, 0)
    wait_chunk(0)
    plsc.subcore_barrier()

    def gather(par, j, r):
        pltpu.async_copy(g_hbm.at[row_r.at[par, j]], bufs[r], gsems[r])

    def gather_wait(r):
        pltpu.make_async_copy(g_hbm.at[row_r.at[0, 0]], bufs[r],
                              gsems[r]).wait()

    def scatter(par, j, r):
        pltpu.async_copy(bufs[r], acc.at[col_r.at[par, j]], ssems[r],
                         add=True)

    def scatter_wait(r):
        pltpu.make_async_copy(bufs[r], acc.at[col_r.at[0, 0]],
                              ssems[r]).wait()

    def scale(par, j, r):
        buf = bufs[r]
        woff = par * wchunk + j * MB

        def sbody(i, carry):
            w16 = w_r[pl.ds(woff + i * 16, 16)]
            for u in range(16):
                wv = w16.at[jnp.full((16,), u, jnp.int32)].get(
                    mode="promise_in_bounds")
                e = i * 16 + u
                for k in range(8):
                    sl = pl.ds(k * 16, 16)
                    buf[e, sl] = buf[e, sl] * wv
            return carry

        lax.fori_loop(0, MB // 16, sbody, 0)

    gather(0, 0, 0)

    def chunk_body(cn, carry):
        par = lax.rem(cn, 2)
        nxt = 1 - par
        for j in range(SB):
            r = j % 2
            gather_wait(r)
            if j == 2:
                @pl.when(cn <= NCH - 2)
                def _():
                    load_chunk(cn + 1, nxt)
            if j >= 1:
                scatter_wait(1 - r)
            else:
                @pl.when(cn >= 1)
                def _():
                    scatter_wait(1 - r)
            if j < SB - 1:
                gather(par, j + 1, 1 - r)
            else:
                @pl.when(cn <= NCH - 2)
                def _():
                    wait_chunk(nxt)
                    gather(nxt, 0, 1 - r)
            scale(par, j, r)
            scatter(par, j, r)
        return carry

    lax.fori_loop(0, NCH, chunk_body, 0)
    scatter_wait((NSUB - 1) % 2)
    plsc.subcore_barrier()
    pltpu.sync_copy(acc.at[pl.ds(tb, ROWS_PER_TILE)],
                    p_hbm.at[pl.ds(tb, ROWS_PER_TILE)])


_BM = 512
_GRID = NPAD // _BM


def _dis(d0_ref, d1_ref):
    deg = d0_ref[...] + d1_ref[...] + 1.0
    return jnp.where(deg > 0, lax.rsqrt(jnp.maximum(deg, 1e-12)), 0.0)


def _dot(a, b):
    return jnp.dot(a, b, preferred_element_type=jnp.float32)


def _enc_body(x_ref, w1, b1, w2, b2, wc, d0, d1, o_ref):
    h = jnp.maximum(_dot(x_ref[...], w1[...]) + b1[...], 0.0)
    h = _dot(h, w2[...]) + b2[...]
    o_ref[...] = _dis(d0, d1) * _dot(h, wc[...])


def _mid_body(p, g, d0, d1, cb, wn, o_ref):
    dis = _dis(d0, d1)
    t = jnp.maximum(dis * (p[...] + g[...]) + cb[...], 0.0)
    o_ref[...] = dis * _dot(t, wn[...])


def _dec_body(p, g, d0, d1, cb, w1, b1, w2, b2, o_ref):
    dis = _dis(d0, d1)
    t = jnp.maximum(dis * (p[...] + g[...]) + cb[...], 0.0)
    t = jnp.maximum(_dot(t, w1[...]) + b1[...], 0.0)
    o = _dot(t, w2[...]) + b2[...]
    o = o - jnp.max(o, axis=1, keepdims=True)
    ex = jnp.exp(o)
    o_ref[...] = ex / jnp.sum(ex, axis=1, keepdims=True)


def _row_spec():
    return pl.BlockSpec((_BM, D), lambda i: (i, 0))


def _col_spec():
    return pl.BlockSpec((_BM, 1), lambda i: (i, 0))


def _wt_spec():
    return pl.BlockSpec((D, D), lambda i: (0, 0))


def _b_spec():
    return pl.BlockSpec((1, D), lambda i: (0, 0))


def _tc_call(body, in_specs, out_dtypes=(jnp.float32,)):
    outs = tuple(jax.ShapeDtypeStruct((NPAD, D), dt) for dt in out_dtypes)
    specs = tuple(_row_spec() for _ in out_dtypes)
    if len(outs) == 1:
        outs, specs = outs[0], specs[0]
    return pl.pallas_call(
        body,
        grid=(_GRID,),
        in_specs=in_specs,
        out_specs=specs,
        out_shape=outs,
    )


def kernel(x, edge_index, edge_weight, enc_w1, enc_b1, enc_w2, enc_b2,
           conv1_w, conv1_b, conv2_w, conv2_b, dec_w1, dec_b1, dec_w2,
           dec_b2):
    def chunk(a, fill):
        a = a.reshape(NW, EW)
        a = jnp.pad(a, ((0, 0), (0, EWP - EW)), constant_values=fill)
        return a.reshape(NW, NBLK, 128)

    def chunk1(a, fill):
        a = a.reshape(NW1, EW1)
        a = jnp.pad(a, ((0, 0), (0, EWP1 - EW1)), constant_values=fill)
        return a.reshape(NW1, NCH, SB, MB)

    rows = chunk(edge_index[0], 0)
    cols = chunk(edge_index[1], 0)
    wgts = chunk(edge_weight, 0.0)
    rows1 = chunk1(edge_index[0], 0)
    cols1 = chunk1(edge_index[1], 0)
    wgts1 = chunk1(edge_weight, 0.0).reshape(NW1, NCH, SB * MB)
    x_p = jnp.pad(x, ((0, NPAD - N), (0, 0)))

    deg0, deg1 = _deg_kernel(wgts, cols)
    d0 = deg0[:, None]
    d1 = deg1[:, None]

    b1r = enc_b1[None, :]
    b2r = enc_b2[None, :]
    c1b = conv1_b[None, :]
    c2b = conv2_b[None, :]
    db1 = dec_b1[None, :]
    db2 = dec_b2[None, :]

    g1 = _tc_call(_enc_body,
                  [_row_spec(), _wt_spec(), _b_spec(), _wt_spec(),
                   _b_spec(), _wt_spec(), _col_spec(), _col_spec()])(
        x_p, enc_w1, b1r, enc_w2, b2r, conv1_w, d0, d1)

    p1 = _mp_kernel(g1, rows1, cols1, wgts1)
    g2 = _tc_call(_mid_body,
                  [_row_spec(), _row_spec(), _col_spec(),
                   _col_spec(), _b_spec(), _wt_spec()])(
        p1, g1, d0, d1, c1b, conv2_w)
    p2 = _mp_kernel(g2, rows1, cols1, wgts1)

    out = _tc_call(_dec_body,
                   [_row_spec(), _row_spec(), _col_spec(), _col_spec(),
                    _b_spec(), _wt_spec(), _b_spec(),
                    _wt_spec(), _b_spec()])(
        p2, g2, d0, d1, c2b, dec_w1, db1, dec_w2, db2)

    return out[:N]


# speedup vs baseline: 1.0704x; 1.0059x over previous
import functools

import jax
import jax.numpy as jnp
from jax import lax
from jax.experimental import pallas as pl
from jax.experimental.pallas import tpu as pltpu
from jax.experimental.pallas import tpu_sc as plsc

N = 10000
E = 320000
D = 128
NPAD = 10240
NW = 32
EW = E // NW
NBLK = 81
EWP = NBLK * 128
NW1 = 16
EW1 = E // NW1
MB = 128
SB = 8
NCH = 20
EWP1 = NCH * SB * MB
NSUB = NCH * SB
ROWS_PER_TILE = NPAD // 16

_MESH = plsc.VectorSubcoreMesh(core_axis_name="c", subcore_axis_name="s")
_MESH1 = plsc.VectorSubcoreMesh(core_axis_name="c", subcore_axis_name="s",
                                num_cores=1)


@functools.partial(
    pl.kernel,
    out_type=(jax.ShapeDtypeStruct((NPAD,), jnp.float32),
              jax.ShapeDtypeStruct((NPAD,), jnp.float32)),
    mesh=_MESH,
    scratch_types=[
        pltpu.VMEM((NBLK, 128), jnp.float32),
        pltpu.VMEM((NBLK, 128), jnp.int32),
        pltpu.VMEM((ROWS_PER_TILE,), jnp.float32),
        pltpu.VMEM_SHARED((NPAD,), jnp.float32),
        pltpu.SemaphoreType.DMA,
    ],
)
def _deg_kernel(w_hbm, col_hbm, deg0_hbm, deg1_hbm, w_v, col_v, zb, acc,
                dsem):
    c = lax.axis_index("c")
    s = lax.axis_index("s")
    wid = s * 2 + c
    tb = s * ROWS_PER_TILE

    for i in range(ROWS_PER_TILE // 16):
        zb[pl.ds(i * 16, 16)] = jnp.zeros((16,), jnp.float32)
    pltpu.sync_copy(zb, acc.at[pl.ds(tb, ROWS_PER_TILE)])
    pltpu.sync_copy(w_hbm.at[wid], w_v)
    pltpu.sync_copy(col_hbm.at[wid], col_v)
    plsc.subcore_barrier()

    def body(j, carry):
        pltpu.async_copy(w_v.at[j], acc.at[col_v.at[j]], dsem, add=True)
        return carry

    lax.fori_loop(0, NBLK, body, 0)

    def drain(j, carry):
        pltpu.make_async_copy(w_v.at[0], acc.at[col_v.at[0]], dsem).wait()
        return carry

    lax.fori_loop(0, NBLK, drain, 0)
    plsc.subcore_barrier()

    @pl.when(c == 0)
    def _():
        pltpu.sync_copy(acc.at[pl.ds(tb, ROWS_PER_TILE)],
                        deg0_hbm.at[pl.ds(tb, ROWS_PER_TILE)])

    @pl.when(c == 1)
    def _():
        pltpu.sync_copy(acc.at[pl.ds(tb, ROWS_PER_TILE)],
                        deg1_hbm.at[pl.ds(tb, ROWS_PER_TILE)])


@functools.partial(
    pl.kernel,
    out_type=jax.ShapeDtypeStruct((NPAD, D), jnp.float32),
    mesh=_MESH1,
    scratch_types=[
        pltpu.VMEM((2, SB, MB), jnp.int32),
        pltpu.VMEM((2, SB, MB), jnp.int32),
        pltpu.VMEM((2 * SB * MB,), jnp.float32),
        pltpu.VMEM((MB, D), jnp.float32),
        pltpu.VMEM((MB, D), jnp.float32),
        pltpu.VMEM_SHARED((NPAD, D), jnp.float32),
        pltpu.SemaphoreType.DMA,
        pltpu.SemaphoreType.DMA,
        pltpu.SemaphoreType.DMA,
        pltpu.SemaphoreType.DMA,
        pltpu.SemaphoreType.DMA,
    ],
)
def _mp_kernel(g_hbm, row_hbm, col_hbm, w_hbm, p_hbm,
               row_r, col_r, w_r, m0, m1, acc,
               gs0, gs1, ss0, ss1, isem):
    s = lax.axis_index("s")
    wid = s
    tb = s * ROWS_PER_TILE
    bufs = (m0, m1)
    gsems = (gs0, gs1)
    ssems = (ss0, ss1)
    wchunk = SB * MB

    def zbody(r, carry):
        for k in range(8):
            m0[r, pl.ds(k * 16, 16)] = jnp.zeros((16,), jnp.float32)
        return carry

    lax.fori_loop(0, MB, zbody, 0)
    for r0 in range(ROWS_PER_TILE // MB):
        pltpu.sync_copy(m0, acc.at[pl.ds(tb + r0 * MB, MB)])

    def load_chunk(cn, par):
        pltpu.async_copy(row_hbm.at[wid, cn], row_r.at[par], isem)
        pltpu.async_copy(col_hbm.at[wid, cn], col_r.at[par], isem)
        pltpu.async_copy(w_hbm.at[wid, cn],
                         w_r.at[pl.ds(par * wchunk, wchunk)], isem)

    def wait_chunk(par):
        pltpu.make_async_copy(row_hbm.at[0, 0], row_r.at[par], isem).wait()
        pltpu.make_async_copy(col_hbm.at[0, 0], col_r.at[par], isem).wait()
        pltpu.make_async_copy(w_hbm.at[0, 0],
                              w_r.at[pl.ds(par * wchunk, wchunk)],
                              isem).wait()

    load_chunk(0, 0)
    wait_chunk(0)
    plsc.subcore_barrier()

    def gather(par, j, r):
        pltpu.async_copy(g_hbm.at[row_r.at[par, j]], bufs[r], gsems[r])

    def gather_wait(r):
        pltpu.make_async_copy(g_hbm.at[row_r.at[0, 0]], bufs[r],
                              gsems[r]).wait()

    def scatter(par, j, r):
        pltpu.async_copy(bufs[r], acc.at[col_r.at[par, j]], ssems[r],
                         add=True)

    def scatter_wait(r):
        pltpu.make_async_copy(bufs[r], acc.at[col_r.at[0, 0]],
                              ssems[r]).wait()

    def scale(par, j, r):
        buf = bufs[r]
        woff = par * wchunk + j * MB

        def sbody(i, carry):
            w16 = w_r[pl.ds(woff + i * 16, 16)]
            for u in range(16):
                wv = w16.at[jnp.full((16,), u, jnp.int32)].get(
                    mode="promise_in_bounds")
                e = i * 16 + u
                for k in range(8):
                    sl = pl.ds(k * 16, 16)
                    buf[e, sl] = buf[e, sl] * wv
            return carry

        lax.fori_loop(0, MB // 16, sbody, 0)

    gather(0, 0, 0)

    def chunk_body(cn, carry):
        par = lax.rem(cn, 2)
        nxt = 1 - par
        for j in range(SB):
            r = j % 2
            gather_wait(r)
            if j == 2:
                @pl.when(cn <= NCH - 2)
                def _():
                    load_chunk(cn + 1, nxt)
            if j >= 1:
                scatter_wait(1 - r)
            else:
                @pl.when(cn >= 1)
                def _():
                    scatter_wait(1 - r)
            if j < SB - 1:
                gather(par, j + 1, 1 - r)
            else:
                @pl.when(cn <= NCH - 2)
                def _():
                    wait_chunk(nxt)
                    gather(nxt, 0, 1 - r)
            scale(par, j, r)
            scatter(par, j, r)
        return carry

    lax.fori_loop(0, NCH, chunk_body, 0)
    scatter_wait((NSUB - 1) % 2)
    plsc.subcore_barrier()
    pltpu.sync_copy(acc.at[pl.ds(tb, ROWS_PER_TILE)],
                    p_hbm.at[pl.ds(tb, ROWS_PER_TILE)])


_BM = 512
_GRID = NPAD // _BM


def _dis(d0_ref, d1_ref):
    deg = d0_ref[...] + d1_ref[...] + 1.0
    return jnp.where(deg > 0, lax.rsqrt(jnp.maximum(deg, 1e-12)), 0.0)


def _dot(a, b):
    return jnp.dot(a, b, preferred_element_type=jnp.float32)


def _enc_body(x_ref, w1, b1, w2, b2, wc, d0, d1, o_ref):
    h = jnp.maximum(_dot(x_ref[...], w1[...]) + b1[...], 0.0)
    h = _dot(h, w2[...]) + b2[...]
    o_ref[...] = _dis(d0, d1) * _dot(h, wc[...])


def _mid_body(p, g, d0, d1, cb, wn, o_ref):
    dis = _dis(d0, d1)
    t = jnp.maximum(dis * (p[...] + g[...]) + cb[...], 0.0)
    o_ref[...] = dis * _dot(t, wn[...])


def _dec_body(p, g, d0, d1, cb, w1, b1, w2, b2, o_ref):
    dis = _dis(d0, d1)
    t = jnp.maximum(dis * (p[...] + g[...]) + cb[...], 0.0)
    t = jnp.maximum(_dot(t, w1[...]) + b1[...], 0.0)
    o = _dot(t, w2[...]) + b2[...]
    o = o - jnp.max(o, axis=1, keepdims=True)
    ex = jnp.exp(o)
    o_ref[...] = ex / jnp.sum(ex, axis=1, keepdims=True)


def _row_spec():
    return pl.BlockSpec((_BM, D), lambda i: (i, 0))


def _col_spec():
    return pl.BlockSpec((_BM, 1), lambda i: (i, 0))


def _wt_spec():
    return pl.BlockSpec((D, D), lambda i: (0, 0))


def _b_spec():
    return pl.BlockSpec((1, D), lambda i: (0, 0))


def _tc_call(body, in_specs, out_dtypes=(jnp.float32,)):
    outs = tuple(jax.ShapeDtypeStruct((NPAD, D), dt) for dt in out_dtypes)
    specs = tuple(_row_spec() for _ in out_dtypes)
    if len(outs) == 1:
        outs, specs = outs[0], specs[0]
    return pl.pallas_call(
        body,
        grid=(_GRID,),
        in_specs=in_specs,
        out_specs=specs,
        out_shape=outs,
    )


def kernel(x, edge_index, edge_weight, enc_w1, enc_b1, enc_w2, enc_b2,
           conv1_w, conv1_b, conv2_w, conv2_b, dec_w1, dec_b1, dec_w2,
           dec_b2):
    def chunk(a, fill):
        a = a.reshape(NW, EW)
        a = jnp.pad(a, ((0, 0), (0, EWP - EW)), constant_values=fill)
        return a.reshape(NW, NBLK, 128)

    def chunk1(a, fill):
        a = a.reshape(NW1, EW1)
        a = jnp.pad(a, ((0, 0), (0, EWP1 - EW1)), constant_values=fill)
        return a.reshape(NW1, NCH, SB, MB)

    rows = chunk(edge_index[0], 0)
    cols = chunk(edge_index[1], 0)
    wgts = chunk(edge_weight, 0.0)
    rows1 = chunk1(edge_index[0], 0)
    cols1 = chunk1(edge_index[1], 0)
    wgts1 = chunk1(edge_weight, 0.0).reshape(NW1, NCH, SB * MB)
    x_p = jnp.pad(x, ((0, NPAD - N), (0, 0)))

    deg0, deg1 = _deg_kernel(wgts, cols)
    d0 = deg0[:, None]
    d1 = deg1[:, None]

    b1r = enc_b1[None, :]
    b2r = enc_b2[None, :]
    c1b = conv1_b[None, :]
    c2b = conv2_b[None, :]
    db1 = dec_b1[None, :]
    db2 = dec_b2[None, :]

    g1 = _tc_call(_enc_body,
                  [_row_spec(), _wt_spec(), _b_spec(), _wt_spec(),
                   _b_spec(), _wt_spec(), _col_spec(), _col_spec()])(
        x_p, enc_w1, b1r, enc_w2, b2r, conv1_w, d0, d1)

    p1 = _mp_kernel(g1, rows1, cols1, wgts1)
    g2 = _tc_call(_mid_body,
                  [_row_spec(), _row_spec(), _col_spec(),
                   _col_spec(), _b_spec(), _wt_spec()])(
        p1, g1, d0, d1, c1b, conv2_w)
    p2 = _mp_kernel(g2, rows1, cols1, wgts1)

    out = _tc_call(_dec_body,
                   [_row_spec(), _row_spec(), _col_spec(), _col_spec(),
                    _b_spec(), _wt_spec(), _b_spec(),
                    _wt_spec(), _b_spec()])(
        p2, g2, d0, d1, c2b, dec_w1, db1, dec_w2, db2)

    return out[:N]
